# baseline (speedup 1.0000x reference)
import functools

import jax
import jax.numpy as jnp
from jax import lax
from jax.experimental import pallas as pl
from jax.experimental.pallas import tpu as pltpu
from jax.experimental.pallas import tpu_sc as plsc

_B, _C, _T = 4, 4, 2048
_VOCAB = 1024
_NROWS = _VOCAB + 1
_LAT = 8
_DM = 512
_NV = 4 * _VOCAB
_NC, _NS = 2, 16
_NW = _NC * _NS
_ROWS = _B * _C * _T
_RPW = _ROWS // _NW
_GCH = 128
_TT = 1024
_NBUF = 2
_NT = _T // _TT


def _sc_gather(emb_table, codes):
    mesh = plsc.VectorSubcoreMesh(core_axis_name="c", subcore_axis_name="s")

    @functools.partial(
        pl.kernel,
        mesh=mesh,
        compiler_params=pltpu.CompilerParams(use_tc_tiling_on_sc=False),
        out_type=jax.ShapeDtypeStruct((_B * _T, _C * _LAT), jnp.float32),
        scratch_types=[
            pltpu.VMEM((_RPW,), jnp.int32),
            pltpu.VMEM((_RPW, _LAT), jnp.float32),
            pltpu.SemaphoreType.DMA,
            pltpu.SemaphoreType.DMA,
        ],
    )
    def k(table_hbm, codes_hbm, out_hbm, idx_v, rows_v, sem, osem):
        wid = lax.axis_index("s") * _NC + lax.axis_index("c")
        base = wid * _RPW
        c_id = (base // _T) % _C
        b_id = base // (_C * _T)
        t0 = base % _T
        pltpu.sync_copy(codes_hbm.at[b_id, c_id, pl.ds(t0, _RPW)], idx_v)
        table_c = table_hbm.at[c_id]
        copies = [
            pltpu.async_copy(
                table_c.at[idx_v.at[pl.ds(j * _GCH, _GCH)]],
                rows_v.at[pl.ds(j * _GCH, _GCH)],
                sem,
            )
            for j in range(_RPW // _GCH)
        ]
        stores = []
        for j, cp in enumerate(copies):
            cp.wait()
            stores.append(
                pltpu.async_copy(
                    rows_v.at[pl.ds(j * _GCH, _GCH)],
                    out_hbm.at[
                        pl.ds(b_id * _T + t0 + j * _GCH, _GCH),
                        pl.ds(c_id * _LAT, _LAT),
                    ],
                    osem,
                )
            )
        for st in stores:
            st.wait()

    return k(emb_table, codes)


def _tc_mlp_kernel(lat_ref, w_in_ref, w_out_ref, out_ref, bufs, sems):
    b = pl.program_id(0)
    t = pl.program_id(1)
    i = b * _NT + t
    k = lax.rem(i, _NBUF)

    @pl.when(i >= _NBUF)
    def _wait_prev():
        ip = i - _NBUF
        bp = ip // _NT
        tp = lax.rem(ip, _NT)
        pltpu.make_async_copy(
            bufs.at[k],
            out_ref.at[bp, :, pl.ds(tp * _TT, _TT)],
            sems.at[k],
        ).wait()

    h = jnp.dot(lat_ref[...], w_in_ref[...], preferred_element_type=jnp.float32)
    h = jax.nn.gelu(h)
    bufs[k] = lax.dot_general(
        w_out_ref[...], h, (((0,), (1,)), ((), ())),
        preferred_element_type=jnp.float32,
    )
    pltpu.make_async_copy(
        bufs.at[k],
        out_ref.at[b, :, pl.ds(t * _TT, _TT)],
        sems.at[k],
    ).start()

    @pl.when(i == _B * _NT - 1)
    def _drain():
        for k2 in range(_NBUF):
            pltpu.make_async_copy(
                bufs.at[k2],
                out_ref.at[b, :, pl.ds(t * _TT, _TT)],
                sems.at[k2],
            ).wait()


def _tc_mlp(latents, w_in, w_out):
    grid = (_B, _NT)
    return pl.pallas_call(
        _tc_mlp_kernel,
        grid=grid,
        in_specs=[
            pl.BlockSpec((_TT, _C * _LAT), lambda b, t: (b * _NT + t, 0)),
            pl.BlockSpec((_C * _LAT, _DM), lambda b, t: (0, 0)),
            pl.BlockSpec((_DM, _NV), lambda b, t: (0, 0)),
        ],
        out_specs=pl.BlockSpec(memory_space=pl.ANY),
        out_shape=jax.ShapeDtypeStruct((_B, _NV, _T), jnp.float32),
        scratch_shapes=[
            pltpu.VMEM((_NBUF, _NV, _TT), jnp.float32),
            pltpu.SemaphoreType.DMA((_NBUF,)),
        ],
    )(latents, w_in, w_out)


def kernel(codes, emb_table, w_in, w_out):
    latents = _sc_gather(emb_table, codes)
    return _tc_mlp(latents, w_in, w_out)


# speedup vs baseline: 1.0309x; 1.0309x over previous
import functools

import jax
import jax.numpy as jnp
from jax import lax
from jax.experimental import pallas as pl
from jax.experimental.pallas import tpu as pltpu
from jax.experimental.pallas import tpu_sc as plsc

_B, _C, _T = 4, 4, 2048
_VOCAB = 1024
_NROWS = _VOCAB + 1
_LAT = 8
_DM = 512
_NV = 4 * _VOCAB
_NC, _NS = 2, 16
_NW = _NC * _NS
_ROWS = _B * _C * _T
_RPW = _ROWS // _NW
_GCH = 128
_TT = 512
_NBUF = 5
_NT = _T // _TT


def _sc_gather(emb_table, codes):
    mesh = plsc.VectorSubcoreMesh(core_axis_name="c", subcore_axis_name="s")

    @functools.partial(
        pl.kernel,
        mesh=mesh,
        compiler_params=pltpu.CompilerParams(use_tc_tiling_on_sc=False),
        out_type=jax.ShapeDtypeStruct((_B * _T // 4, 128), jnp.float32),
        scratch_types=[
            pltpu.VMEM((_RPW,), jnp.int32),
            pltpu.VMEM((_RPW, _LAT), jnp.float32),
            pltpu.SemaphoreType.DMA,
            pltpu.SemaphoreType.DMA,
        ],
    )
    def k(table_hbm, codes_hbm, out_hbm, idx_v, rows_v, sem, osem):
        wid = lax.axis_index("s") * _NC + lax.axis_index("c")
        base = wid * _RPW
        c_id = (base // _T) % _C
        b_id = base // (_C * _T)
        t0 = base % _T
        pltpu.sync_copy(codes_hbm.at[b_id, c_id, pl.ds(t0, _RPW)], idx_v)
        table_c = table_hbm.at[c_id]
        copies = [
            pltpu.async_copy(
                table_c.at[idx_v.at[pl.ds(j * _GCH, _GCH)]],
                rows_v.at[pl.ds(j * _GCH, _GCH)],
                sem,
            )
            for j in range(_RPW // _GCH)
        ]
        stores = []
        for j, cp in enumerate(copies):
            cp.wait()
            g0 = b_id * _T + t0 + j * _GCH
            q0 = (g0 // 512) * 128
            slot = (g0 % 512) // _GCH
            stores.append(
                pltpu.async_copy(
                    rows_v.at[pl.ds(j * _GCH, _GCH)],
                    out_hbm.at[
                        pl.ds(q0, _GCH),
                        pl.ds(slot * 32 + c_id * _LAT, _LAT),
                    ],
                    osem,
                )
            )
        for st in stores:
            st.wait()

    return k(emb_table, codes)


def _tc_mlp_kernel(lat_ref, w_in_ref, w_out_ref, out_ref, bufs, sems):
    b = pl.program_id(0)
    t = pl.program_id(1)
    i = b * _NT + t
    k = lax.rem(i, _NBUF)

    @pl.when(i >= _NBUF)
    def _wait_prev():
        ip = i - _NBUF
        bp = ip // _NT
        tp = lax.rem(ip, _NT)
        pltpu.make_async_copy(
            bufs.at[k],
            out_ref.at[bp, :, pl.ds(tp * _TT, _TT)],
            sems.at[k],
        ).wait()

    v = lat_ref[...]
    h4 = jnp.dot(v, w_in_ref[...], preferred_element_type=jnp.float32)
    h4 = jax.nn.gelu(h4)
    w_out = w_out_ref[...]
    pair01 = jnp.concatenate([h4[:, 0:_DM], h4[:, _DM:2 * _DM]], axis=0)
    pair23 = jnp.concatenate([h4[:, 2 * _DM:3 * _DM], h4[:, 3 * _DM:]], axis=0)
    bufs[k, :, 0:256] = lax.dot_general(
        w_out, pair01, (((0,), (1,)), ((), ())),
        preferred_element_type=jnp.float32,
    )
    bufs[k, :, 256:512] = lax.dot_general(
        w_out, pair23, (((0,), (1,)), ((), ())),
        preferred_element_type=jnp.float32,
    )
    pltpu.make_async_copy(
        bufs.at[k],
        out_ref.at[b, :, pl.ds(t * _TT, _TT)],
        sems.at[k],
    ).start()

    @pl.when(i == _B * _NT - 1)
    def _drain():
        for k2 in range(_NBUF):
            pltpu.make_async_copy(
                bufs.at[k2],
                out_ref.at[b, :, pl.ds(t * _TT, _TT)],
                sems.at[k2],
            ).wait()


def _tc_mlp(latents, w_in, w_out):
    grid = (_B, _NT)
    return pl.pallas_call(
        _tc_mlp_kernel,
        grid=grid,
        in_specs=[
            pl.BlockSpec((_TT // 4, 128), lambda b, t: (b * _NT + t, 0)),
            pl.BlockSpec((128, 4 * _DM), lambda b, t: (0, 0)),
            pl.BlockSpec((_DM, _NV), lambda b, t: (0, 0)),
        ],
        out_specs=pl.BlockSpec(memory_space=pl.ANY),
        out_shape=jax.ShapeDtypeStruct((_B, _NV, _T), jnp.float32),
        scratch_shapes=[
            pltpu.VMEM((_NBUF, _NV, _TT), jnp.float32),
            pltpu.SemaphoreType.DMA((_NBUF,)),
        ],
    )(latents, w_in, w_out)


def kernel(codes, emb_table, w_in, w_out):
    w_in_exp = (
        jnp.eye(4, dtype=w_in.dtype)[:, None, :, None]
        * w_in[None, :, None, :]
    ).reshape(4 * _C * _LAT, 4 * _DM)
    latents = _sc_gather(emb_table, codes)
    return _tc_mlp(latents, w_in_exp, w_out)


# speedup vs baseline: 1.0384x; 1.0073x over previous
import functools

import jax
import jax.numpy as jnp
from jax import lax
from jax.experimental import pallas as pl
from jax.experimental.pallas import tpu as pltpu
from jax.experimental.pallas import tpu_sc as plsc

_B, _C, _T = 4, 4, 2048
_VOCAB = 1024
_NROWS = _VOCAB + 1
_LAT = 8
_DM = 512
_NV = 4 * _VOCAB
_NC, _NS = 2, 16
_NW = _NC * _NS
_ROWS = _B * _C * _T
_RPW = _ROWS // _NW
_GCH = 128
_TT = 512
_NBUF = 5
_NT = _T // _TT


def _sc_gather(emb_table, codes):
    mesh = plsc.VectorSubcoreMesh(core_axis_name="c", subcore_axis_name="s")

    @functools.partial(
        pl.kernel,
        mesh=mesh,
        compiler_params=pltpu.CompilerParams(use_tc_tiling_on_sc=False),
        out_type=jax.ShapeDtypeStruct((_B * _T // 4, 128), jnp.float32),
        scratch_types=[
            pltpu.VMEM((_RPW,), jnp.int32),
            pltpu.VMEM((_RPW, _LAT), jnp.float32),
            pltpu.SemaphoreType.DMA,
            pltpu.SemaphoreType.DMA,
        ],
    )
    def k(table_hbm, codes_hbm, out_hbm, idx_v, rows_v, sem, osem):
        wid = lax.axis_index("s") * _NC + lax.axis_index("c")
        base = wid * _RPW
        c_id = (base // _T) % _C
        b_id = base // (_C * _T)
        t0 = base % _T
        pltpu.sync_copy(codes_hbm.at[b_id, c_id, pl.ds(t0, _RPW)], idx_v)
        table_c = table_hbm.at[c_id]
        copies = [
            pltpu.async_copy(
                table_c.at[idx_v.at[pl.ds(j * _GCH, _GCH)]],
                rows_v.at[pl.ds(j * _GCH, _GCH)],
                sem,
            )
            for j in range(_RPW // _GCH)
        ]
        stores = []
        for j, cp in enumerate(copies):
            cp.wait()
            g0 = b_id * _T + t0 + j * _GCH
            q0 = (g0 // 512) * 128
            slot = (g0 % 512) // _GCH
            stores.append(
                pltpu.async_copy(
                    rows_v.at[pl.ds(j * _GCH, _GCH)],
                    out_hbm.at[
                        pl.ds(q0, _GCH),
                        pl.ds(slot * 32 + c_id * _LAT, _LAT),
                    ],
                    osem,
                )
            )
        for st in stores:
            st.wait()

    return k(emb_table, codes)


def _tc_mlp_kernel(lat_ref, w_in_ref, w_out_ref, out_ref, bufs, sems):
    b = pl.program_id(0)
    t = pl.program_id(1)
    i = b * _NT + t
    k = lax.rem(i, _NBUF)

    @pl.when(i >= _NBUF)
    def _wait_prev():
        ip = i - _NBUF
        bp = ip // _NT
        tp = lax.rem(ip, _NT)
        pltpu.make_async_copy(
            bufs.at[k],
            out_ref.at[bp, :, pl.ds(tp * _TT, _TT)],
            sems.at[k],
        ).wait()

    v = lat_ref[...]
    h4 = jnp.dot(v, w_in_ref[...], preferred_element_type=jnp.float32)
    h4 = jax.nn.gelu(h4)
    h = jnp.concatenate(
        [h4[:, s * _DM:(s + 1) * _DM] for s in range(4)], axis=0
    )
    bufs[k] = lax.dot_general(
        w_out_ref[...], h, (((0,), (1,)), ((), ())),
        preferred_element_type=jnp.float32,
    )
    pltpu.make_async_copy(
        bufs.at[k],
        out_ref.at[b, :, pl.ds(t * _TT, _TT)],
        sems.at[k],
    ).start()

    @pl.when(i == _B * _NT - 1)
    def _drain():
        for k2 in range(_NBUF):
            pltpu.make_async_copy(
                bufs.at[k2],
                out_ref.at[b, :, pl.ds(t * _TT, _TT)],
                sems.at[k2],
            ).wait()


def _tc_mlp(latents, w_in, w_out):
    grid = (_B, _NT)
    return pl.pallas_call(
        _tc_mlp_kernel,
        grid=grid,
        in_specs=[
            pl.BlockSpec((_TT // 4, 128), lambda b, t: (b * _NT + t, 0)),
            pl.BlockSpec((128, 4 * _DM), lambda b, t: (0, 0)),
            pl.BlockSpec((_DM, _NV), lambda b, t: (0, 0)),
        ],
        out_specs=pl.BlockSpec(memory_space=pl.ANY),
        out_shape=jax.ShapeDtypeStruct((_B, _NV, _T), jnp.float32),
        scratch_shapes=[
            pltpu.VMEM((_NBUF, _NV, _TT), jnp.float32),
            pltpu.SemaphoreType.DMA((_NBUF,)),
        ],
    )(latents, w_in, w_out)


def kernel(codes, emb_table, w_in, w_out):
    w_in_exp = (
        jnp.eye(4, dtype=w_in.dtype)[:, None, :, None]
        * w_in[None, :, None, :]
    ).reshape(4 * _C * _LAT, 4 * _DM)
    latents = _sc_gather(emb_table, codes)
    return _tc_mlp(latents, w_in_exp, w_out)
